# baseline (speedup 1.0000x reference)
import functools

import jax
import jax.numpy as jnp
from jax import lax
from jax.experimental import pallas as pl
from jax.experimental.pallas import tpu as pltpu
from jax.experimental.pallas import tpu_sc as plsc

N_NODES = 10000
N_EDGES = 320000
N_IN = 128
N_HID = 256
N_LAT = 128

NC = 2
NS = 16
HALF = N_HID // 2
NPAD = 10240
PAD_NODE = 10050
CH = 40
ECT = 327680 // NS
NCHUNK = ECT // CH
DCH = 128
E_PAD = 327680
DROWS = E_PAD // (NC * NS * DCH)
ACC_ROWS = 10112
SEG = ACC_ROWS // NS


def _zero_vmem_2d(ref, nrows, ncols):
    def row(r, _):
        def col(j, _):
            ref[r, pl.ds(j * 16, 16)] = jnp.zeros((16,), jnp.float32)
            return 0
        return lax.fori_loop(0, ncols // 16, col, 0)
    lax.fori_loop(0, nrows, row, 0)


def _deg_body(dst_hbm, out_hbm, didx, ones_v, zb_v, dacc):
    c = lax.axis_index("c")
    s = lax.axis_index("s")
    seg = NPAD // NS

    def zchunk(i, _):
        zb_v[pl.ds(i * 16, 16)] = jnp.zeros((16,), jnp.float32)
        return 0
    lax.fori_loop(0, seg // 16, zchunk, 0)

    def ochunk(i, _):
        ones_v[pl.ds(i * 16, 16)] = jnp.ones((16,), jnp.float32)
        return 0
    lax.fori_loop(0, DCH // 16, ochunk, 0)

    r0 = (c * NS + s) * DROWS
    pltpu.sync_copy(dst_hbm.at[pl.ds(r0, DROWS)], didx)
    pltpu.sync_copy(zb_v, dacc.at[pl.ds(s * seg, seg)])
    plsc.subcore_barrier()

    def chunk(j, _):
        pltpu.sync_copy(ones_v, dacc.at[didx.at[j]], add=True)
        return 0
    lax.fori_loop(0, DROWS, chunk, 0)

    plsc.subcore_barrier()
    pltpu.sync_copy(dacc.at[pl.ds(s * seg, seg)], out_hbm.at[c, pl.ds(s * seg, seg)])


def _spmm_body(src_hbm, dst_hbm, ta_hbm, tb_hbm, zeros_hbm, out_hbm,
               sidx0, sidx1, didx0, didx1, rows0, rows1, acc, semg0, semg1):
    c = lax.axis_index("c")
    s = lax.axis_index("s")

    pltpu.sync_copy(zeros_hbm, acc.at[pl.ds(s * SEG, SEG)])
    plsc.subcore_barrier()

    base = s * ECT

    def fetch_src(j, buf):
        pltpu.sync_copy(src_hbm.at[pl.ds(base + j * CH, CH)], buf)

    def fetch_dst(j, buf):
        pltpu.sync_copy(dst_hbm.at[pl.ds(base + j * CH, CH)], buf)

    def start_gather(buf, idx, sem):
        @pl.when(c == 0)
        def _():
            pltpu.make_async_copy(ta_hbm.at[idx], buf, sem).start()

        @pl.when(c == 1)
        def _():
            pltpu.make_async_copy(tb_hbm.at[idx], buf, sem).start()

    def wait_gather(buf, idx, sem):
        pltpu.make_async_copy(ta_hbm.at[idx], buf, sem).wait()

    fetch_src(0, sidx0)
    start_gather(rows0, sidx0, semg0)
    fetch_src(1, sidx1)
    start_gather(rows1, sidx1, semg1)

    def body(jj, _):
        j = jj * 2
        fetch_dst(j, didx0)
        wait_gather(rows0, sidx0, semg0)
        pltpu.sync_copy(rows0, acc.at[didx0], add=True)

        @pl.when(j + 2 < NCHUNK)
        def _():
            fetch_src(j + 2, sidx0)
            start_gather(rows0, sidx0, semg0)

        fetch_dst(j + 1, didx1)
        wait_gather(rows1, sidx1, semg1)
        pltpu.sync_copy(rows1, acc.at[didx1], add=True)

        @pl.when(j + 3 < NCHUNK)
        def _():
            fetch_src(j + 3, sidx1)
            start_gather(rows1, sidx1, semg1)
        return 0
    lax.fori_loop(0, NCHUNK // 2, body, 0)

    plsc.subcore_barrier()
    pltpu.sync_copy(acc.at[pl.ds(s * SEG, SEG)], out_hbm.at[c, pl.ds(s * SEG, SEG)])


@functools.lru_cache(maxsize=None)
def _sc_kernels():
    mesh = plsc.VectorSubcoreMesh(
        core_axis_name="c", subcore_axis_name="s", num_cores=NC, num_subcores=NS
    )
    deg = pl.kernel(
        _deg_body,
        out_type=jax.ShapeDtypeStruct((NC, NPAD), jnp.float32),
        mesh=mesh,
        scratch_types=[
            pltpu.VMEM((DROWS, DCH), jnp.int32),
            pltpu.VMEM((DCH,), jnp.float32),
            pltpu.VMEM((NPAD // NS,), jnp.float32),
            pltpu.VMEM_SHARED((NPAD,), jnp.float32),
        ],
    )
    spmm = pl.kernel(
        _spmm_body,
        out_type=jax.ShapeDtypeStruct((NC, NPAD, HALF), jnp.float32),
        mesh=mesh,
        scratch_types=[
            pltpu.VMEM((CH,), jnp.int32),
            pltpu.VMEM((CH,), jnp.int32),
            pltpu.VMEM((CH,), jnp.int32),
            pltpu.VMEM((CH,), jnp.int32),
            pltpu.VMEM((CH, HALF), jnp.float32),
            pltpu.VMEM((CH, HALF), jnp.float32),
            pltpu.VMEM_SHARED((ACC_ROWS, HALF), jnp.float32),
            pltpu.SemaphoreType.DMA,
            pltpu.SemaphoreType.DMA,
        ],
    )
    return deg, spmm


_RB = 1024


def _k1_body(x_ref, w_ref, b_ref, d0_ref, d1_ref, dinv_ref, ha_ref, hb_ref):
    deg = d0_ref[...] + d1_ref[...] + 1.0
    dinv = lax.rsqrt(deg)
    h = jnp.dot(x_ref[...], w_ref[...], preferred_element_type=jnp.float32)
    h = jnp.maximum(h + b_ref[...], 0.0)
    hp = h * dinv
    dinv_ref[...] = dinv
    ha_ref[...] = hp[:, :HALF]
    hb_ref[...] = hp[:, HALF:]


def _k1(x, w, b, d0, d1):
    g = NPAD // _RB
    return pl.pallas_call(
        _k1_body,
        grid=(g,),
        in_specs=[
            pl.BlockSpec((_RB, N_IN), lambda i: (i, 0)),
            pl.BlockSpec((N_IN, N_HID), lambda i: (0, 0)),
            pl.BlockSpec((1, N_HID), lambda i: (0, 0)),
            pl.BlockSpec((_RB, 1), lambda i: (i, 0)),
            pl.BlockSpec((_RB, 1), lambda i: (i, 0)),
        ],
        out_specs=[
            pl.BlockSpec((_RB, 1), lambda i: (i, 0)),
            pl.BlockSpec((_RB, HALF), lambda i: (i, 0)),
            pl.BlockSpec((_RB, HALF), lambda i: (i, 0)),
        ],
        out_shape=[
            jax.ShapeDtypeStruct((NPAD, 1), jnp.float32),
            jax.ShapeDtypeStruct((NPAD, HALF), jnp.float32),
            jax.ShapeDtypeStruct((NPAD, HALF), jnp.float32),
        ],
    )(x, w, b, d0, d1)


def _k3_body(sa_ref, sb_ref, ha_ref, hb_ref, dinv_ref, w_ref, b_ref, oa_ref, ob_ref):
    dinv = dinv_ref[...]
    h1 = jnp.concatenate(
        [(sa_ref[...] + ha_ref[...]) * dinv, (sb_ref[...] + hb_ref[...]) * dinv],
        axis=1,
    )
    h2 = jnp.dot(h1, w_ref[...], preferred_element_type=jnp.float32)
    h2 = jnp.maximum(h2 + b_ref[...], 0.0) * dinv
    oa_ref[...] = h2[:, :HALF]
    ob_ref[...] = h2[:, HALF:]


def _k3(sa, sb, ha, hb, dinv, w, b):
    g = NPAD // _RB
    half_spec = pl.BlockSpec((_RB, HALF), lambda i: (i, 0))
    return pl.pallas_call(
        _k3_body,
        grid=(g,),
        in_specs=[
            half_spec, half_spec, half_spec, half_spec,
            pl.BlockSpec((_RB, 1), lambda i: (i, 0)),
            pl.BlockSpec((N_HID, N_HID), lambda i: (0, 0)),
            pl.BlockSpec((1, N_HID), lambda i: (0, 0)),
        ],
        out_specs=[half_spec, half_spec],
        out_shape=[
            jax.ShapeDtypeStruct((NPAD, HALF), jnp.float32),
            jax.ShapeDtypeStruct((NPAD, HALF), jnp.float32),
        ],
    )(sa, sb, ha, hb, dinv, w, b)


def _k4_body(sa_ref, sb_ref, ha_ref, hb_ref, dinv_ref, wm_ref, bm_ref,
             wl_ref, bl_ref, mu_ref, ls_ref):
    dinv = dinv_ref[...]
    h2 = jnp.concatenate(
        [(sa_ref[...] + ha_ref[...]) * dinv, (sb_ref[...] + hb_ref[...]) * dinv],
        axis=1,
    )
    mu_ref[...] = jnp.dot(h2, wm_ref[...], preferred_element_type=jnp.float32) + bm_ref[...]
    ls_ref[...] = jnp.dot(h2, wl_ref[...], preferred_element_type=jnp.float32) + bl_ref[...]


def _k4(sa, sb, ha, hb, dinv, wm, bm, wl, bl):
    g = NPAD // _RB
    half_spec = pl.BlockSpec((_RB, HALF), lambda i: (i, 0))
    lat_spec = pl.BlockSpec((_RB, N_LAT), lambda i: (i, 0))
    return pl.pallas_call(
        _k4_body,
        grid=(g,),
        in_specs=[
            half_spec, half_spec, half_spec, half_spec,
            pl.BlockSpec((_RB, 1), lambda i: (i, 0)),
            pl.BlockSpec((N_HID, N_LAT), lambda i: (0, 0)),
            pl.BlockSpec((1, N_LAT), lambda i: (0, 0)),
            pl.BlockSpec((N_HID, N_LAT), lambda i: (0, 0)),
            pl.BlockSpec((1, N_LAT), lambda i: (0, 0)),
        ],
        out_specs=[lat_spec, lat_spec],
        out_shape=[
            jax.ShapeDtypeStruct((NPAD, N_LAT), jnp.float32),
            jax.ShapeDtypeStruct((NPAD, N_LAT), jnp.float32),
        ],
    )(sa, sb, ha, hb, dinv, wm, bm, wl, bl)


def kernel(x, edge_index, W_fc, b_fc, W1, b1, W_mu, b_mu, W_ls, b_ls):
    _deg_sc, _spmm_sc = _sc_kernels()

    pad = jnp.full((E_PAD - N_EDGES,), PAD_NODE, dtype=edge_index.dtype)
    src = jnp.concatenate([edge_index[0], pad])
    dst = jnp.concatenate([edge_index[1], pad])
    dst2d = dst.reshape(E_PAD // DCH, DCH)
    x_pad = jnp.concatenate(
        [x, jnp.zeros((NPAD - N_NODES, N_IN), dtype=x.dtype)], axis=0
    )

    d = _deg_sc(dst2d)
    d0 = d[0].reshape(NPAD, 1)
    d1 = d[1].reshape(NPAD, 1)

    zseg = jnp.zeros((SEG, HALF), dtype=jnp.float32)
    dinv, hpa, hpb = _k1(x_pad, W_fc, b_fc.reshape(1, -1), d0, d1)
    s1 = _spmm_sc(src, dst, hpa, hpb, zseg)
    h2pa, h2pb = _k3(s1[0], s1[1], hpa, hpb, dinv, W1, b1.reshape(1, -1))
    s2 = _spmm_sc(src, dst, h2pa, h2pb, zseg)
    mu, ls = _k4(s2[0], s2[1], h2pa, h2pb, dinv,
                 W_mu, b_mu.reshape(1, -1), W_ls, b_ls.reshape(1, -1))
    return (mu[:N_NODES], ls[:N_NODES])


# speedup vs baseline: 1.7016x; 1.7016x over previous
import functools

import jax
import jax.numpy as jnp
from jax import lax
from jax.experimental import pallas as pl
from jax.experimental.pallas import tpu as pltpu
from jax.experimental.pallas import tpu_sc as plsc

N_NODES = 10000
N_EDGES = 320000
N_IN = 128
N_HID = 256
N_LAT = 128

NC = 2
NS = 16
HALF = N_HID // 2
NPAD = 10240
PAD_NODE = 10050
CH = 96
NCHUNK = 2 * (-(-320000 // (NS * CH * 2)))
ECT = NCHUNK * CH
E_SPMM = NS * ECT
DCH = 128
E_PAD = 327680
DROWS = E_PAD // (NC * NS * DCH)
ACC_ROWS = 10112
SEG = ACC_ROWS // NS


def _zero_vmem_2d(ref, nrows, ncols):
    def row(r, _):
        def col(j, _):
            ref[r, pl.ds(j * 16, 16)] = jnp.zeros((16,), jnp.float32)
            return 0
        return lax.fori_loop(0, ncols // 16, col, 0)
    lax.fori_loop(0, nrows, row, 0)


def _deg_body(dst_hbm, out_hbm, didx, ones_v, zb_v, dacc):
    c = lax.axis_index("c")
    s = lax.axis_index("s")
    seg = NPAD // NS

    def zchunk(i, _):
        zb_v[pl.ds(i * 16, 16)] = jnp.zeros((16,), jnp.float32)
        return 0
    lax.fori_loop(0, seg // 16, zchunk, 0)

    def ochunk(i, _):
        ones_v[pl.ds(i * 16, 16)] = jnp.ones((16,), jnp.float32)
        return 0
    lax.fori_loop(0, DCH // 16, ochunk, 0)

    r0 = (c * NS + s) * DROWS
    pltpu.sync_copy(dst_hbm.at[pl.ds(r0, DROWS)], didx)
    pltpu.sync_copy(zb_v, dacc.at[pl.ds(s * seg, seg)])
    plsc.subcore_barrier()

    def chunk(j, _):
        pltpu.sync_copy(ones_v, dacc.at[didx.at[j]], add=True)
        return 0
    lax.fori_loop(0, DROWS, chunk, 0)

    plsc.subcore_barrier()
    pltpu.sync_copy(dacc.at[pl.ds(s * seg, seg)], out_hbm.at[c, pl.ds(s * seg, seg)])


def _spmm_body(src_hbm, dst_hbm, ta_hbm, tb_hbm, zeros_hbm, out_hbm,
               sidx0, sidx1, didx0, didx1, rows0, rows1, acc, semg0, semg1):
    c = lax.axis_index("c")
    s = lax.axis_index("s")

    pltpu.sync_copy(zeros_hbm, acc.at[pl.ds(s * SEG, SEG)])
    plsc.subcore_barrier()

    base = s * ECT

    def fetch_src(j, buf):
        pltpu.sync_copy(src_hbm.at[pl.ds(base + j * CH, CH)], buf)

    def fetch_dst(j, buf):
        pltpu.sync_copy(dst_hbm.at[pl.ds(base + j * CH, CH)], buf)

    def start_gather(buf, idx, sem):
        @pl.when(c == 0)
        def _():
            pltpu.make_async_copy(ta_hbm.at[idx], buf, sem).start()

        @pl.when(c == 1)
        def _():
            pltpu.make_async_copy(tb_hbm.at[idx], buf, sem).start()

    def wait_gather(buf, idx, sem):
        pltpu.make_async_copy(ta_hbm.at[idx], buf, sem).wait()

    fetch_src(0, sidx0)
    start_gather(rows0, sidx0, semg0)
    fetch_src(1, sidx1)
    start_gather(rows1, sidx1, semg1)

    def body(jj, _):
        j = jj * 2
        fetch_dst(j, didx0)
        wait_gather(rows0, sidx0, semg0)
        pltpu.sync_copy(rows0, acc.at[didx0], add=True)

        @pl.when(j + 2 < NCHUNK)
        def _():
            fetch_src(j + 2, sidx0)
            start_gather(rows0, sidx0, semg0)

        fetch_dst(j + 1, didx1)
        wait_gather(rows1, sidx1, semg1)
        pltpu.sync_copy(rows1, acc.at[didx1], add=True)

        @pl.when(j + 3 < NCHUNK)
        def _():
            fetch_src(j + 3, sidx1)
            start_gather(rows1, sidx1, semg1)
        return 0
    lax.fori_loop(0, NCHUNK // 2, body, 0)

    plsc.subcore_barrier()
    pltpu.sync_copy(acc.at[pl.ds(s * SEG, SEG)], out_hbm.at[c, pl.ds(s * SEG, SEG)])


@functools.lru_cache(maxsize=None)
def _sc_kernels():
    mesh = plsc.VectorSubcoreMesh(
        core_axis_name="c", subcore_axis_name="s", num_cores=NC, num_subcores=NS
    )
    deg = pl.kernel(
        _deg_body,
        out_type=jax.ShapeDtypeStruct((NC, NPAD), jnp.float32),
        mesh=mesh,
        scratch_types=[
            pltpu.VMEM((DROWS, DCH), jnp.int32),
            pltpu.VMEM((DCH,), jnp.float32),
            pltpu.VMEM((NPAD // NS,), jnp.float32),
            pltpu.VMEM_SHARED((NPAD,), jnp.float32),
        ],
    )
    spmm = pl.kernel(
        _spmm_body,
        out_type=jax.ShapeDtypeStruct((NC, NPAD, HALF), jnp.float32),
        mesh=mesh,
        scratch_types=[
            pltpu.VMEM((CH,), jnp.int32),
            pltpu.VMEM((CH,), jnp.int32),
            pltpu.VMEM((CH,), jnp.int32),
            pltpu.VMEM((CH,), jnp.int32),
            pltpu.VMEM((CH, HALF), jnp.float32),
            pltpu.VMEM((CH, HALF), jnp.float32),
            pltpu.VMEM_SHARED((ACC_ROWS, HALF), jnp.float32),
            pltpu.SemaphoreType.DMA,
            pltpu.SemaphoreType.DMA,
        ],
    )
    return deg, spmm


_RB = 1024


def _k1_body(x_ref, w_ref, b_ref, d0_ref, d1_ref, dinv_ref, ha_ref, hb_ref):
    deg = d0_ref[...] + d1_ref[...] + 1.0
    dinv = lax.rsqrt(deg)
    h = jnp.dot(x_ref[...], w_ref[...], preferred_element_type=jnp.float32)
    h = jnp.maximum(h + b_ref[...], 0.0)
    hp = h * dinv
    dinv_ref[...] = dinv
    ha_ref[...] = hp[:, :HALF]
    hb_ref[...] = hp[:, HALF:]


def _k1(x, w, b, d0, d1):
    g = NPAD // _RB
    return pl.pallas_call(
        _k1_body,
        grid=(g,),
        in_specs=[
            pl.BlockSpec((_RB, N_IN), lambda i: (i, 0)),
            pl.BlockSpec((N_IN, N_HID), lambda i: (0, 0)),
            pl.BlockSpec((1, N_HID), lambda i: (0, 0)),
            pl.BlockSpec((_RB, 1), lambda i: (i, 0)),
            pl.BlockSpec((_RB, 1), lambda i: (i, 0)),
        ],
        out_specs=[
            pl.BlockSpec((_RB, 1), lambda i: (i, 0)),
            pl.BlockSpec((_RB, HALF), lambda i: (i, 0)),
            pl.BlockSpec((_RB, HALF), lambda i: (i, 0)),
        ],
        out_shape=[
            jax.ShapeDtypeStruct((NPAD, 1), jnp.float32),
            jax.ShapeDtypeStruct((NPAD, HALF), jnp.float32),
            jax.ShapeDtypeStruct((NPAD, HALF), jnp.float32),
        ],
    )(x, w, b, d0, d1)


def _k3_body(sa_ref, sb_ref, ha_ref, hb_ref, dinv_ref, w_ref, b_ref, oa_ref, ob_ref):
    dinv = dinv_ref[...]
    h1 = jnp.concatenate(
        [(sa_ref[...] + ha_ref[...]) * dinv, (sb_ref[...] + hb_ref[...]) * dinv],
        axis=1,
    )
    h2 = jnp.dot(h1, w_ref[...], preferred_element_type=jnp.float32)
    h2 = jnp.maximum(h2 + b_ref[...], 0.0) * dinv
    oa_ref[...] = h2[:, :HALF]
    ob_ref[...] = h2[:, HALF:]


def _k3(sa, sb, ha, hb, dinv, w, b):
    g = NPAD // _RB
    half_spec = pl.BlockSpec((_RB, HALF), lambda i: (i, 0))
    return pl.pallas_call(
        _k3_body,
        grid=(g,),
        in_specs=[
            half_spec, half_spec, half_spec, half_spec,
            pl.BlockSpec((_RB, 1), lambda i: (i, 0)),
            pl.BlockSpec((N_HID, N_HID), lambda i: (0, 0)),
            pl.BlockSpec((1, N_HID), lambda i: (0, 0)),
        ],
        out_specs=[half_spec, half_spec],
        out_shape=[
            jax.ShapeDtypeStruct((NPAD, HALF), jnp.float32),
            jax.ShapeDtypeStruct((NPAD, HALF), jnp.float32),
        ],
    )(sa, sb, ha, hb, dinv, w, b)


def _k4_body(sa_ref, sb_ref, ha_ref, hb_ref, dinv_ref, wm_ref, bm_ref,
             wl_ref, bl_ref, mu_ref, ls_ref):
    dinv = dinv_ref[...]
    h2 = jnp.concatenate(
        [(sa_ref[...] + ha_ref[...]) * dinv, (sb_ref[...] + hb_ref[...]) * dinv],
        axis=1,
    )
    mu_ref[...] = jnp.dot(h2, wm_ref[...], preferred_element_type=jnp.float32) + bm_ref[...]
    ls_ref[...] = jnp.dot(h2, wl_ref[...], preferred_element_type=jnp.float32) + bl_ref[...]


def _k4(sa, sb, ha, hb, dinv, wm, bm, wl, bl):
    g = NPAD // _RB
    half_spec = pl.BlockSpec((_RB, HALF), lambda i: (i, 0))
    lat_spec = pl.BlockSpec((_RB, N_LAT), lambda i: (i, 0))
    return pl.pallas_call(
        _k4_body,
        grid=(g,),
        in_specs=[
            half_spec, half_spec, half_spec, half_spec,
            pl.BlockSpec((_RB, 1), lambda i: (i, 0)),
            pl.BlockSpec((N_HID, N_LAT), lambda i: (0, 0)),
            pl.BlockSpec((1, N_LAT), lambda i: (0, 0)),
            pl.BlockSpec((N_HID, N_LAT), lambda i: (0, 0)),
            pl.BlockSpec((1, N_LAT), lambda i: (0, 0)),
        ],
        out_specs=[lat_spec, lat_spec],
        out_shape=[
            jax.ShapeDtypeStruct((NPAD, N_LAT), jnp.float32),
            jax.ShapeDtypeStruct((NPAD, N_LAT), jnp.float32),
        ],
    )(sa, sb, ha, hb, dinv, wm, bm, wl, bl)


def kernel(x, edge_index, W_fc, b_fc, W1, b1, W_mu, b_mu, W_ls, b_ls):
    _deg_sc, _spmm_sc = _sc_kernels()

    pad_s = jnp.full((E_SPMM - N_EDGES,), PAD_NODE, dtype=edge_index.dtype)
    src = jnp.concatenate([edge_index[0], pad_s])
    dst = jnp.concatenate([edge_index[1], pad_s])
    pad_d = jnp.full((E_PAD - N_EDGES,), PAD_NODE, dtype=edge_index.dtype)
    dst2d = jnp.concatenate([edge_index[1], pad_d]).reshape(E_PAD // DCH, DCH)
    x_pad = jnp.concatenate(
        [x, jnp.zeros((NPAD - N_NODES, N_IN), dtype=x.dtype)], axis=0
    )

    d = _deg_sc(dst2d)
    d0 = d[0].reshape(NPAD, 1)
    d1 = d[1].reshape(NPAD, 1)

    zseg = jnp.zeros((SEG, HALF), dtype=jnp.float32)
    dinv, hpa, hpb = _k1(x_pad, W_fc, b_fc.reshape(1, -1), d0, d1)
    s1 = _spmm_sc(src, dst, hpa, hpb, zseg)
    h2pa, h2pb = _k3(s1[0], s1[1], hpa, hpb, dinv, W1, b1.reshape(1, -1))
    s2 = _spmm_sc(src, dst, h2pa, h2pb, zseg)
    mu, ls = _k4(s2[0], s2[1], h2pa, h2pb, dinv,
                 W_mu, b_mu.reshape(1, -1), W_ls, b_ls.reshape(1, -1))
    return (mu[:N_NODES], ls[:N_NODES])


# speedup vs baseline: 1.9626x; 1.1534x over previous
import functools

import jax
import jax.numpy as jnp
from jax import lax
from jax.experimental import pallas as pl
from jax.experimental.pallas import tpu as pltpu
from jax.experimental.pallas import tpu_sc as plsc

N_NODES = 10000
N_EDGES = 320000
N_IN = 128
N_HID = 256
N_LAT = 128

NC = 2
NS = 16
HALF = N_HID // 2
NPAD = 10240
PAD_NODE = 10050
CH = 88
NCHUNK = 2 * (-(-320000 // (NS * CH * 2)))
ECT = NCHUNK * CH
E_SPMM = NS * ECT
DCH = 128
E_PAD = 327680
DROWS = E_PAD // (NC * NS * DCH)
ACC_ROWS = 10112
SEG = ACC_ROWS // NS


def _zero_vmem_2d(ref, nrows, ncols):
    def row(r, _):
        def col(j, _):
            ref[r, pl.ds(j * 16, 16)] = jnp.zeros((16,), jnp.float32)
            return 0
        return lax.fori_loop(0, ncols // 16, col, 0)
    lax.fori_loop(0, nrows, row, 0)


def _deg_body(dst_hbm, out_hbm, didx, ones_v, zb_v, dacc):
    c = lax.axis_index("c")
    s = lax.axis_index("s")
    seg = NPAD // NS

    def zchunk(i, _):
        zb_v[pl.ds(i * 16, 16)] = jnp.zeros((16,), jnp.float32)
        return 0
    lax.fori_loop(0, seg // 16, zchunk, 0)

    def ochunk(i, _):
        ones_v[pl.ds(i * 16, 16)] = jnp.ones((16,), jnp.float32)
        return 0
    lax.fori_loop(0, DCH // 16, ochunk, 0)

    r0 = (c * NS + s) * DROWS
    pltpu.sync_copy(dst_hbm.at[pl.ds(r0, DROWS)], didx)
    pltpu.sync_copy(zb_v, dacc.at[pl.ds(s * seg, seg)])
    plsc.subcore_barrier()

    def chunk(j, _):
        pltpu.sync_copy(ones_v, dacc.at[didx.at[j]], add=True)
        return 0
    lax.fori_loop(0, DROWS, chunk, 0)

    plsc.subcore_barrier()
    pltpu.sync_copy(dacc.at[pl.ds(s * seg, seg)], out_hbm.at[c, pl.ds(s * seg, seg)])


def _spmm_body(src_hbm, dst_hbm, ta_hbm, tb_hbm, zeros_hbm, out_hbm,
               sidx0, sidx1, didx0, didx1, rows0, rows1, acc, semg0, semg1):
    c = lax.axis_index("c")
    s = lax.axis_index("s")

    pltpu.sync_copy(zeros_hbm, acc.at[pl.ds(s * SEG, SEG)])
    plsc.subcore_barrier()

    base = s * ECT

    def fetch_src(j, buf):
        pltpu.sync_copy(src_hbm.at[pl.ds(base + j * CH, CH)], buf)

    def fetch_dst(j, buf):
        pltpu.sync_copy(dst_hbm.at[pl.ds(base + j * CH, CH)], buf)

    def start_gather(buf, idx, sem):
        @pl.when(c == 0)
        def _():
            pltpu.make_async_copy(ta_hbm.at[idx], buf, sem).start()

        @pl.when(c == 1)
        def _():
            pltpu.make_async_copy(tb_hbm.at[idx], buf, sem).start()

    def wait_gather(buf, idx, sem):
        pltpu.make_async_copy(ta_hbm.at[idx], buf, sem).wait()

    fetch_src(0, sidx0)
    start_gather(rows0, sidx0, semg0)
    fetch_src(1, sidx1)
    start_gather(rows1, sidx1, semg1)

    def body(jj, _):
        j = jj * 2
        fetch_dst(j, didx0)
        wait_gather(rows0, sidx0, semg0)
        pltpu.sync_copy(rows0, acc.at[didx0], add=True)

        @pl.when(j + 2 < NCHUNK)
        def _():
            fetch_src(j + 2, sidx0)
            start_gather(rows0, sidx0, semg0)

        fetch_dst(j + 1, didx1)
        wait_gather(rows1, sidx1, semg1)
        pltpu.sync_copy(rows1, acc.at[didx1], add=True)

        @pl.when(j + 3 < NCHUNK)
        def _():
            fetch_src(j + 3, sidx1)
            start_gather(rows1, sidx1, semg1)
        return 0
    lax.fori_loop(0, NCHUNK // 2, body, 0)

    plsc.subcore_barrier()
    pltpu.sync_copy(acc.at[pl.ds(s * SEG, SEG)], out_hbm.at[c, pl.ds(s * SEG, SEG)])


@functools.lru_cache(maxsize=None)
def _sc_kernels():
    mesh = plsc.VectorSubcoreMesh(
        core_axis_name="c", subcore_axis_name="s", num_cores=NC, num_subcores=NS
    )
    deg = pl.kernel(
        _deg_body,
        out_type=jax.ShapeDtypeStruct((NC, NPAD), jnp.float32),
        mesh=mesh,
        scratch_types=[
            pltpu.VMEM((DROWS, DCH), jnp.int32),
            pltpu.VMEM((DCH,), jnp.float32),
            pltpu.VMEM((NPAD // NS,), jnp.float32),
            pltpu.VMEM_SHARED((NPAD,), jnp.float32),
        ],
    )
    spmm = pl.kernel(
        _spmm_body,
        out_type=jax.ShapeDtypeStruct((NC, NPAD, HALF), jnp.float32),
        mesh=mesh,
        scratch_types=[
            pltpu.VMEM((CH,), jnp.int32),
            pltpu.VMEM((CH,), jnp.int32),
            pltpu.VMEM((CH,), jnp.int32),
            pltpu.VMEM((CH,), jnp.int32),
            pltpu.VMEM((CH, HALF), jnp.float32),
            pltpu.VMEM((CH, HALF), jnp.float32),
            pltpu.VMEM_SHARED((ACC_ROWS, HALF), jnp.float32),
            pltpu.SemaphoreType.DMA,
            pltpu.SemaphoreType.DMA,
        ],
    )
    return deg, spmm


_RB = 1024


def _k1_body(x_ref, w_ref, b_ref, d0_ref, d1_ref, dinv_ref, ha_ref, hb_ref):
    deg = d0_ref[...] + d1_ref[...] + 1.0
    dinv = lax.rsqrt(deg)
    h = jnp.dot(x_ref[...], w_ref[...], preferred_element_type=jnp.float32)
    h = jnp.maximum(h + b_ref[...], 0.0)
    hp = h * dinv
    dinv_ref[...] = dinv
    ha_ref[...] = hp[:, :HALF]
    hb_ref[...] = hp[:, HALF:]


def _k1(x, w, b, d0, d1):
    g = NPAD // _RB
    return pl.pallas_call(
        _k1_body,
        grid=(g,),
        in_specs=[
            pl.BlockSpec((_RB, N_IN), lambda i: (i, 0)),
            pl.BlockSpec((N_IN, N_HID), lambda i: (0, 0)),
            pl.BlockSpec((1, N_HID), lambda i: (0, 0)),
            pl.BlockSpec((_RB, 1), lambda i: (i, 0)),
            pl.BlockSpec((_RB, 1), lambda i: (i, 0)),
        ],
        out_specs=[
            pl.BlockSpec((_RB, 1), lambda i: (i, 0)),
            pl.BlockSpec((_RB, HALF), lambda i: (i, 0)),
            pl.BlockSpec((_RB, HALF), lambda i: (i, 0)),
        ],
        out_shape=[
            jax.ShapeDtypeStruct((NPAD, 1), jnp.float32),
            jax.ShapeDtypeStruct((NPAD, HALF), jnp.float32),
            jax.ShapeDtypeStruct((NPAD, HALF), jnp.float32),
        ],
    )(x, w, b, d0, d1)


def _k3_body(sa_ref, sb_ref, ha_ref, hb_ref, dinv_ref, w_ref, b_ref, oa_ref, ob_ref):
    dinv = dinv_ref[...]
    h1 = jnp.concatenate(
        [(sa_ref[...] + ha_ref[...]) * dinv, (sb_ref[...] + hb_ref[...]) * dinv],
        axis=1,
    )
    h2 = jnp.dot(h1, w_ref[...], preferred_element_type=jnp.float32)
    h2 = jnp.maximum(h2 + b_ref[...], 0.0) * dinv
    oa_ref[...] = h2[:, :HALF]
    ob_ref[...] = h2[:, HALF:]


def _k3(sa, sb, ha, hb, dinv, w, b):
    g = NPAD // _RB
    half_spec = pl.BlockSpec((_RB, HALF), lambda i: (i, 0))
    return pl.pallas_call(
        _k3_body,
        grid=(g,),
        in_specs=[
            half_spec, half_spec, half_spec, half_spec,
            pl.BlockSpec((_RB, 1), lambda i: (i, 0)),
            pl.BlockSpec((N_HID, N_HID), lambda i: (0, 0)),
            pl.BlockSpec((1, N_HID), lambda i: (0, 0)),
        ],
        out_specs=[half_spec, half_spec],
        out_shape=[
            jax.ShapeDtypeStruct((NPAD, HALF), jnp.float32),
            jax.ShapeDtypeStruct((NPAD, HALF), jnp.float32),
        ],
    )(sa, sb, ha, hb, dinv, w, b)


def _k4_body(sa_ref, sb_ref, ha_ref, hb_ref, dinv_ref, wm_ref, bm_ref,
             wl_ref, bl_ref, mu_ref, ls_ref):
    dinv = dinv_ref[...]
    h2 = jnp.concatenate(
        [(sa_ref[...] + ha_ref[...]) * dinv, (sb_ref[...] + hb_ref[...]) * dinv],
        axis=1,
    )
    mu_ref[...] = jnp.dot(h2, wm_ref[...], preferred_element_type=jnp.float32) + bm_ref[...]
    ls_ref[...] = jnp.dot(h2, wl_ref[...], preferred_element_type=jnp.float32) + bl_ref[...]


def _k4(sa, sb, ha, hb, dinv, wm, bm, wl, bl):
    g = NPAD // _RB
    half_spec = pl.BlockSpec((_RB, HALF), lambda i: (i, 0))
    lat_spec = pl.BlockSpec((_RB, N_LAT), lambda i: (i, 0))
    return pl.pallas_call(
        _k4_body,
        grid=(g,),
        in_specs=[
            half_spec, half_spec, half_spec, half_spec,
            pl.BlockSpec((_RB, 1), lambda i: (i, 0)),
            pl.BlockSpec((N_HID, N_LAT), lambda i: (0, 0)),
            pl.BlockSpec((1, N_LAT), lambda i: (0, 0)),
            pl.BlockSpec((N_HID, N_LAT), lambda i: (0, 0)),
            pl.BlockSpec((1, N_LAT), lambda i: (0, 0)),
        ],
        out_specs=[lat_spec, lat_spec],
        out_shape=[
            jax.ShapeDtypeStruct((NPAD, N_LAT), jnp.float32),
            jax.ShapeDtypeStruct((NPAD, N_LAT), jnp.float32),
        ],
    )(sa, sb, ha, hb, dinv, wm, bm, wl, bl)


def kernel(x, edge_index, W_fc, b_fc, W1, b1, W_mu, b_mu, W_ls, b_ls):
    _deg_sc, _spmm_sc = _sc_kernels()

    pad_s = jnp.full((E_SPMM - N_EDGES,), PAD_NODE, dtype=edge_index.dtype)
    src = jnp.concatenate([edge_index[0], pad_s])
    dst = jnp.concatenate([edge_index[1], pad_s])
    pad_d = jnp.full((E_PAD - N_EDGES,), PAD_NODE, dtype=edge_index.dtype)
    dst2d = jnp.concatenate([edge_index[1], pad_d]).reshape(E_PAD // DCH, DCH)
    x_pad = jnp.concatenate(
        [x, jnp.zeros((NPAD - N_NODES, N_IN), dtype=x.dtype)], axis=0
    )

    d = _deg_sc(dst2d)
    d0 = d[0].reshape(NPAD, 1)
    d1 = d[1].reshape(NPAD, 1)

    zseg = jnp.zeros((SEG, HALF), dtype=jnp.float32)
    dinv, hpa, hpb = _k1(x_pad, W_fc, b_fc.reshape(1, -1), d0, d1)
    s1 = _spmm_sc(src, dst, hpa, hpb, zseg)
    h2pa, h2pb = _k3(s1[0], s1[1], hpa, hpb, dinv, W1, b1.reshape(1, -1))
    s2 = _spmm_sc(src, dst, h2pa, h2pb, zseg)
    mu, ls = _k4(s2[0], s2[1], h2pa, h2pb, dinv,
                 W_mu, b_mu.reshape(1, -1), W_ls, b_ls.reshape(1, -1))
    return (mu[:N_NODES], ls[:N_NODES])


# speedup vs baseline: 2.0515x; 1.0453x over previous
import functools

import jax
import jax.numpy as jnp
from jax import lax
from jax.experimental import pallas as pl
from jax.experimental.pallas import tpu as pltpu
from jax.experimental.pallas import tpu_sc as plsc

N_NODES = 10000
N_EDGES = 320000
N_IN = 128
N_HID = 256
N_LAT = 128

NC = 2
NS = 16
HALF = N_HID // 2
NPAD = 10240
PAD_NODE = 10050
CH = 80
NCHUNK = 2 * (-(-320000 // (NS * CH * 2)))
ECT = NCHUNK * CH
E_SPMM = NS * ECT
DCH = 128
E_PAD = 327680
DROWS = E_PAD // (NC * NS * DCH)
ACC_ROWS = 10112
SEG = ACC_ROWS // NS


def _zero_vmem_2d(ref, nrows, ncols):
    def row(r, _):
        def col(j, _):
            ref[r, pl.ds(j * 16, 16)] = jnp.zeros((16,), jnp.float32)
            return 0
        return lax.fori_loop(0, ncols // 16, col, 0)
    lax.fori_loop(0, nrows, row, 0)


def _deg_body(dst_hbm, out_hbm, didx, ones_v, zb_v, dacc):
    c = lax.axis_index("c")
    s = lax.axis_index("s")
    seg = NPAD // NS

    def zchunk(i, _):
        zb_v[pl.ds(i * 16, 16)] = jnp.zeros((16,), jnp.float32)
        return 0
    lax.fori_loop(0, seg // 16, zchunk, 0)

    def ochunk(i, _):
        ones_v[pl.ds(i * 16, 16)] = jnp.ones((16,), jnp.float32)
        return 0
    lax.fori_loop(0, DCH // 16, ochunk, 0)

    r0 = (c * NS + s) * DROWS
    pltpu.sync_copy(dst_hbm.at[pl.ds(r0, DROWS)], didx)
    pltpu.sync_copy(zb_v, dacc.at[pl.ds(s * seg, seg)])
    plsc.subcore_barrier()

    def chunk(j, _):
        pltpu.sync_copy(ones_v, dacc.at[didx.at[j]], add=True)
        return 0
    lax.fori_loop(0, DROWS, chunk, 0)

    plsc.subcore_barrier()
    pltpu.sync_copy(dacc.at[pl.ds(s * seg, seg)], out_hbm.at[c, pl.ds(s * seg, seg)])


def _spmm_body(src_hbm, dst_hbm, ta_hbm, tb_hbm, zeros_hbm, out_hbm,
               sidx0, sidx1, didx0, didx1, rows0, rows1, acc, semg0, semg1):
    c = lax.axis_index("c")
    s = lax.axis_index("s")

    pltpu.sync_copy(zeros_hbm, acc.at[pl.ds(s * SEG, SEG)])
    plsc.subcore_barrier()

    base = s * ECT

    def fetch_src(j, buf):
        pltpu.sync_copy(src_hbm.at[pl.ds(base + j * CH, CH)], buf)

    def fetch_dst(j, buf):
        pltpu.sync_copy(dst_hbm.at[pl.ds(base + j * CH, CH)], buf)

    def start_gather(buf, idx, sem):
        @pl.when(c == 0)
        def _():
            pltpu.make_async_copy(ta_hbm.at[idx], buf, sem).start()

        @pl.when(c == 1)
        def _():
            pltpu.make_async_copy(tb_hbm.at[idx], buf, sem).start()

    def wait_gather(buf, idx, sem):
        pltpu.make_async_copy(ta_hbm.at[idx], buf, sem).wait()

    fetch_src(0, sidx0)
    start_gather(rows0, sidx0, semg0)
    fetch_src(1, sidx1)
    start_gather(rows1, sidx1, semg1)

    def body(jj, _):
        j = jj * 2
        fetch_dst(j, didx0)
        wait_gather(rows0, sidx0, semg0)
        pltpu.sync_copy(rows0, acc.at[didx0], add=True)

        @pl.when(j + 2 < NCHUNK)
        def _():
            fetch_src(j + 2, sidx0)
            start_gather(rows0, sidx0, semg0)

        fetch_dst(j + 1, didx1)
        wait_gather(rows1, sidx1, semg1)
        pltpu.sync_copy(rows1, acc.at[didx1], add=True)

        @pl.when(j + 3 < NCHUNK)
        def _():
            fetch_src(j + 3, sidx1)
            start_gather(rows1, sidx1, semg1)
        return 0
    lax.fori_loop(0, NCHUNK // 2, body, 0)

    plsc.subcore_barrier()
    pltpu.sync_copy(acc.at[pl.ds(s * SEG, SEG)], out_hbm.at[c, pl.ds(s * SEG, SEG)])


@functools.lru_cache(maxsize=None)
def _sc_kernels():
    mesh = plsc.VectorSubcoreMesh(
        core_axis_name="c", subcore_axis_name="s", num_cores=NC, num_subcores=NS
    )
    deg = pl.kernel(
        _deg_body,
        out_type=jax.ShapeDtypeStruct((NC, NPAD), jnp.float32),
        mesh=mesh,
        scratch_types=[
            pltpu.VMEM((DROWS, DCH), jnp.int32),
            pltpu.VMEM((DCH,), jnp.float32),
            pltpu.VMEM((NPAD // NS,), jnp.float32),
            pltpu.VMEM_SHARED((NPAD,), jnp.float32),
        ],
    )
    spmm = pl.kernel(
        _spmm_body,
        out_type=jax.ShapeDtypeStruct((NC, NPAD, HALF), jnp.float32),
        mesh=mesh,
        scratch_types=[
            pltpu.VMEM((CH,), jnp.int32),
            pltpu.VMEM((CH,), jnp.int32),
            pltpu.VMEM((CH,), jnp.int32),
            pltpu.VMEM((CH,), jnp.int32),
            pltpu.VMEM((CH, HALF), jnp.float32),
            pltpu.VMEM((CH, HALF), jnp.float32),
            pltpu.VMEM_SHARED((ACC_ROWS, HALF), jnp.float32),
            pltpu.SemaphoreType.DMA,
            pltpu.SemaphoreType.DMA,
        ],
    )
    return deg, spmm


_RB = 1024


def _k1_body(x_ref, w_ref, b_ref, d0_ref, d1_ref, dinv_ref, ha_ref, hb_ref):
    deg = d0_ref[...] + d1_ref[...] + 1.0
    dinv = lax.rsqrt(deg)
    h = jnp.dot(x_ref[...], w_ref[...], preferred_element_type=jnp.float32)
    h = jnp.maximum(h + b_ref[...], 0.0)
    hp = h * dinv
    dinv_ref[...] = dinv
    ha_ref[...] = hp[:, :HALF]
    hb_ref[...] = hp[:, HALF:]


def _k1(x, w, b, d0, d1):
    g = NPAD // _RB
    return pl.pallas_call(
        _k1_body,
        grid=(g,),
        in_specs=[
            pl.BlockSpec((_RB, N_IN), lambda i: (i, 0)),
            pl.BlockSpec((N_IN, N_HID), lambda i: (0, 0)),
            pl.BlockSpec((1, N_HID), lambda i: (0, 0)),
            pl.BlockSpec((_RB, 1), lambda i: (i, 0)),
            pl.BlockSpec((_RB, 1), lambda i: (i, 0)),
        ],
        out_specs=[
            pl.BlockSpec((_RB, 1), lambda i: (i, 0)),
            pl.BlockSpec((_RB, HALF), lambda i: (i, 0)),
            pl.BlockSpec((_RB, HALF), lambda i: (i, 0)),
        ],
        out_shape=[
            jax.ShapeDtypeStruct((NPAD, 1), jnp.float32),
            jax.ShapeDtypeStruct((NPAD, HALF), jnp.float32),
            jax.ShapeDtypeStruct((NPAD, HALF), jnp.float32),
        ],
    )(x, w, b, d0, d1)


def _k3_body(sa_ref, sb_ref, ha_ref, hb_ref, dinv_ref, w_ref, b_ref, oa_ref, ob_ref):
    dinv = dinv_ref[...]
    h1 = jnp.concatenate(
        [(sa_ref[...] + ha_ref[...]) * dinv, (sb_ref[...] + hb_ref[...]) * dinv],
        axis=1,
    )
    h2 = jnp.dot(h1, w_ref[...], preferred_element_type=jnp.float32)
    h2 = jnp.maximum(h2 + b_ref[...], 0.0) * dinv
    oa_ref[...] = h2[:, :HALF]
    ob_ref[...] = h2[:, HALF:]


def _k3(sa, sb, ha, hb, dinv, w, b):
    g = NPAD // _RB
    half_spec = pl.BlockSpec((_RB, HALF), lambda i: (i, 0))
    return pl.pallas_call(
        _k3_body,
        grid=(g,),
        in_specs=[
            half_spec, half_spec, half_spec, half_spec,
            pl.BlockSpec((_RB, 1), lambda i: (i, 0)),
            pl.BlockSpec((N_HID, N_HID), lambda i: (0, 0)),
            pl.BlockSpec((1, N_HID), lambda i: (0, 0)),
        ],
        out_specs=[half_spec, half_spec],
        out_shape=[
            jax.ShapeDtypeStruct((NPAD, HALF), jnp.float32),
            jax.ShapeDtypeStruct((NPAD, HALF), jnp.float32),
        ],
    )(sa, sb, ha, hb, dinv, w, b)


def _k4_body(sa_ref, sb_ref, ha_ref, hb_ref, dinv_ref, wm_ref, bm_ref,
             wl_ref, bl_ref, mu_ref, ls_ref):
    dinv = dinv_ref[...]
    h2 = jnp.concatenate(
        [(sa_ref[...] + ha_ref[...]) * dinv, (sb_ref[...] + hb_ref[...]) * dinv],
        axis=1,
    )
    mu_ref[...] = jnp.dot(h2, wm_ref[...], preferred_element_type=jnp.float32) + bm_ref[...]
    ls_ref[...] = jnp.dot(h2, wl_ref[...], preferred_element_type=jnp.float32) + bl_ref[...]


def _k4(sa, sb, ha, hb, dinv, wm, bm, wl, bl):
    g = NPAD // _RB
    half_spec = pl.BlockSpec((_RB, HALF), lambda i: (i, 0))
    lat_spec = pl.BlockSpec((_RB, N_LAT), lambda i: (i, 0))
    return pl.pallas_call(
        _k4_body,
        grid=(g,),
        in_specs=[
            half_spec, half_spec, half_spec, half_spec,
            pl.BlockSpec((_RB, 1), lambda i: (i, 0)),
            pl.BlockSpec((N_HID, N_LAT), lambda i: (0, 0)),
            pl.BlockSpec((1, N_LAT), lambda i: (0, 0)),
            pl.BlockSpec((N_HID, N_LAT), lambda i: (0, 0)),
            pl.BlockSpec((1, N_LAT), lambda i: (0, 0)),
        ],
        out_specs=[lat_spec, lat_spec],
        out_shape=[
            jax.ShapeDtypeStruct((NPAD, N_LAT), jnp.float32),
            jax.ShapeDtypeStruct((NPAD, N_LAT), jnp.float32),
        ],
    )(sa, sb, ha, hb, dinv, wm, bm, wl, bl)


def kernel(x, edge_index, W_fc, b_fc, W1, b1, W_mu, b_mu, W_ls, b_ls):
    _deg_sc, _spmm_sc = _sc_kernels()

    pad_s = jnp.full((E_SPMM - N_EDGES,), PAD_NODE, dtype=edge_index.dtype)
    src = jnp.concatenate([edge_index[0], pad_s])
    dst = jnp.concatenate([edge_index[1], pad_s])
    pad_d = jnp.full((E_PAD - N_EDGES,), PAD_NODE, dtype=edge_index.dtype)
    dst2d = jnp.concatenate([edge_index[1], pad_d]).reshape(E_PAD // DCH, DCH)
    x_pad = jnp.concatenate(
        [x, jnp.zeros((NPAD - N_NODES, N_IN), dtype=x.dtype)], axis=0
    )

    d = _deg_sc(dst2d)
    d0 = d[0].reshape(NPAD, 1)
    d1 = d[1].reshape(NPAD, 1)

    zseg = jnp.zeros((SEG, HALF), dtype=jnp.float32)
    dinv, hpa, hpb = _k1(x_pad, W_fc, b_fc.reshape(1, -1), d0, d1)
    s1 = _spmm_sc(src, dst, hpa, hpb, zseg)
    h2pa, h2pb = _k3(s1[0], s1[1], hpa, hpb, dinv, W1, b1.reshape(1, -1))
    s2 = _spmm_sc(src, dst, h2pa, h2pb, zseg)
    mu, ls = _k4(s2[0], s2[1], h2pa, h2pb, dinv,
                 W_mu, b_mu.reshape(1, -1), W_ls, b_ls.reshape(1, -1))
    return (mu[:N_NODES], ls[:N_NODES])


# speedup vs baseline: 2.1094x; 1.0282x over previous
import functools

import jax
import jax.numpy as jnp
from jax import lax
from jax.experimental import pallas as pl
from jax.experimental.pallas import tpu as pltpu
from jax.experimental.pallas import tpu_sc as plsc

N_NODES = 10000
N_EDGES = 320000
N_IN = 128
N_HID = 256
N_LAT = 128

NC = 2
NS = 16
HALF = N_HID // 2
NPAD = 10240
PAD_NODE = 10050
CH = 80
NCHUNK = 2 * (-(-320000 // (NS * CH * 2)))
ECT = NCHUNK * CH
E_SPMM = NS * ECT
DCH = 128
E_PAD = 327680
DROWS = E_PAD // (NC * NS * DCH)
ACC_ROWS = 10112
SEG = ACC_ROWS // NS


def _zero_vmem_2d(ref, nrows, ncols):
    def row(r, _):
        def col(j, _):
            ref[r, pl.ds(j * 16, 16)] = jnp.zeros((16,), jnp.float32)
            return 0
        return lax.fori_loop(0, ncols // 16, col, 0)
    lax.fori_loop(0, nrows, row, 0)


def _deg_body(dst_hbm, out_hbm, didx, ones_v, zb_v, dacc):
    c = lax.axis_index("c")
    s = lax.axis_index("s")
    seg = NPAD // NS

    def zchunk(i, _):
        zb_v[pl.ds(i * 16, 16)] = jnp.zeros((16,), jnp.float32)
        return 0
    lax.fori_loop(0, seg // 16, zchunk, 0)

    def ochunk(i, _):
        ones_v[pl.ds(i * 16, 16)] = jnp.ones((16,), jnp.float32)
        return 0
    lax.fori_loop(0, DCH // 16, ochunk, 0)

    r0 = (c * NS + s) * DROWS
    pltpu.sync_copy(dst_hbm.at[pl.ds(r0, DROWS)], didx)
    pltpu.sync_copy(zb_v, dacc.at[pl.ds(s * seg, seg)])
    plsc.subcore_barrier()

    def chunk(j, _):
        pltpu.sync_copy(ones_v, dacc.at[didx.at[j]], add=True)
        return 0
    lax.fori_loop(0, DROWS, chunk, 0)

    plsc.subcore_barrier()
    pltpu.sync_copy(dacc.at[pl.ds(s * seg, seg)], out_hbm.at[c, pl.ds(s * seg, seg)])


def _spmm_body(src_hbm, dst_hbm, ta_hbm, tb_hbm, zeros_hbm, out_hbm,
               sidx0, sidx1, didx0, didx1, rows0, rows1, acc, semg0, semg1):
    c = lax.axis_index("c")
    s = lax.axis_index("s")

    pltpu.sync_copy(zeros_hbm, acc.at[pl.ds(s * SEG, SEG)])
    plsc.subcore_barrier()

    base = s * ECT

    def fetch_src(j, buf):
        pltpu.sync_copy(src_hbm.at[pl.ds(base + j * CH, CH)], buf)

    def fetch_dst(j, buf):
        pltpu.sync_copy(dst_hbm.at[pl.ds(base + j * CH, CH)], buf)

    def start_gather(buf, idx, sem):
        @pl.when(c == 0)
        def _():
            pltpu.make_async_copy(ta_hbm.at[idx], buf, sem).start()

        @pl.when(c == 1)
        def _():
            pltpu.make_async_copy(tb_hbm.at[idx], buf, sem).start()

    def wait_gather(buf, idx, sem):
        pltpu.make_async_copy(ta_hbm.at[idx], buf, sem).wait()

    fetch_src(0, sidx0)
    start_gather(rows0, sidx0, semg0)
    fetch_src(1, sidx1)
    start_gather(rows1, sidx1, semg1)

    def body(jj, _):
        j = jj * 2
        fetch_dst(j, didx0)
        wait_gather(rows0, sidx0, semg0)
        pltpu.sync_copy(rows0, acc.at[didx0], add=True)

        @pl.when(j + 2 < NCHUNK)
        def _():
            fetch_src(j + 2, sidx0)
            start_gather(rows0, sidx0, semg0)

        fetch_dst(j + 1, didx1)
        wait_gather(rows1, sidx1, semg1)
        pltpu.sync_copy(rows1, acc.at[didx1], add=True)

        @pl.when(j + 3 < NCHUNK)
        def _():
            fetch_src(j + 3, sidx1)
            start_gather(rows1, sidx1, semg1)
        return 0
    lax.fori_loop(0, NCHUNK // 2, body, 0)

    plsc.subcore_barrier()
    pltpu.sync_copy(acc.at[pl.ds(s * SEG, SEG)], out_hbm.at[c, pl.ds(s * SEG, SEG)])


@functools.lru_cache(maxsize=None)
def _sc_kernels():
    mesh = plsc.VectorSubcoreMesh(
        core_axis_name="c", subcore_axis_name="s", num_cores=NC, num_subcores=NS
    )
    deg = pl.kernel(
        _deg_body,
        out_type=jax.ShapeDtypeStruct((NC, NPAD), jnp.float32),
        mesh=mesh,
        scratch_types=[
            pltpu.VMEM((DROWS, DCH), jnp.int32),
            pltpu.VMEM((DCH,), jnp.float32),
            pltpu.VMEM((NPAD // NS,), jnp.float32),
            pltpu.VMEM_SHARED((NPAD,), jnp.float32),
        ],
    )
    spmm = pl.kernel(
        _spmm_body,
        out_type=jax.ShapeDtypeStruct((NC, NPAD, HALF), jnp.float32),
        mesh=mesh,
        scratch_types=[
            pltpu.VMEM((CH,), jnp.int32),
            pltpu.VMEM((CH,), jnp.int32),
            pltpu.VMEM((CH,), jnp.int32),
            pltpu.VMEM((CH,), jnp.int32),
            pltpu.VMEM((CH, HALF), jnp.float32),
            pltpu.VMEM((CH, HALF), jnp.float32),
            pltpu.VMEM_SHARED((ACC_ROWS, HALF), jnp.float32),
            pltpu.SemaphoreType.DMA,
            pltpu.SemaphoreType.DMA,
        ],
    )
    return deg, spmm


_RB = 1024


def _k1_body(x_ref, w_ref, b_ref, d0_ref, d1_ref, dinv_ref, ha_ref, hb_ref):
    deg = d0_ref[...] + d1_ref[...] + 1.0
    dinv = lax.rsqrt(deg)
    h = jnp.dot(x_ref[...], w_ref[...], preferred_element_type=jnp.float32)
    h = jnp.maximum(h + b_ref[...], 0.0)
    hp = h * dinv
    dinv_ref[...] = dinv
    ha_ref[...] = hp[:, :HALF]
    hb_ref[...] = hp[:, HALF:]


def _k1(x, w, b, d0, d1):
    g = NPAD // _RB
    return pl.pallas_call(
        _k1_body,
        grid=(g,),
        in_specs=[
            pl.BlockSpec((_RB, N_IN), lambda i: (i, 0)),
            pl.BlockSpec((N_IN, N_HID), lambda i: (0, 0)),
            pl.BlockSpec((1, N_HID), lambda i: (0, 0)),
            pl.BlockSpec((_RB, 1), lambda i: (i, 0)),
            pl.BlockSpec((_RB, 1), lambda i: (i, 0)),
        ],
        out_specs=[
            pl.BlockSpec((_RB, 1), lambda i: (i, 0)),
            pl.BlockSpec((_RB, HALF), lambda i: (i, 0)),
            pl.BlockSpec((_RB, HALF), lambda i: (i, 0)),
        ],
        out_shape=[
            jax.ShapeDtypeStruct((NPAD, 1), jnp.float32),
            jax.ShapeDtypeStruct((NPAD, HALF), jnp.float32),
            jax.ShapeDtypeStruct((NPAD, HALF), jnp.float32),
        ],
    )(x, w, b, d0, d1)


def _k3_body(sa_ref, sb_ref, ha_ref, hb_ref, dinv_ref, w_ref, b_ref, oa_ref, ob_ref):
    dinv = dinv_ref[...]
    h1 = jnp.concatenate(
        [(sa_ref[0] + ha_ref[...]) * dinv, (sb_ref[0] + hb_ref[...]) * dinv],
        axis=1,
    )
    h2 = jnp.dot(h1, w_ref[...], preferred_element_type=jnp.float32)
    h2 = jnp.maximum(h2 + b_ref[...], 0.0) * dinv
    oa_ref[...] = h2[:, :HALF]
    ob_ref[...] = h2[:, HALF:]


def _k3(sagg, ha, hb, dinv, w, b):
    g = NPAD // _RB
    half_spec = pl.BlockSpec((_RB, HALF), lambda i: (i, 0))
    sa_spec = pl.BlockSpec((1, _RB, HALF), lambda i: (0, i, 0))
    sb_spec = pl.BlockSpec((1, _RB, HALF), lambda i: (1, i, 0))
    return pl.pallas_call(
        _k3_body,
        grid=(g,),
        in_specs=[
            sa_spec, sb_spec, half_spec, half_spec,
            pl.BlockSpec((_RB, 1), lambda i: (i, 0)),
            pl.BlockSpec((N_HID, N_HID), lambda i: (0, 0)),
            pl.BlockSpec((1, N_HID), lambda i: (0, 0)),
        ],
        out_specs=[half_spec, half_spec],
        out_shape=[
            jax.ShapeDtypeStruct((NPAD, HALF), jnp.float32),
            jax.ShapeDtypeStruct((NPAD, HALF), jnp.float32),
        ],
    )(sagg, sagg, ha, hb, dinv, w, b)


def _k4_body(sa_ref, sb_ref, ha_ref, hb_ref, dinv_ref, wm_ref, bm_ref,
             wl_ref, bl_ref, mu_ref, ls_ref):
    dinv = dinv_ref[...]
    h2 = jnp.concatenate(
        [(sa_ref[0] + ha_ref[...]) * dinv, (sb_ref[0] + hb_ref[...]) * dinv],
        axis=1,
    )
    mu_ref[...] = jnp.dot(h2, wm_ref[...], preferred_element_type=jnp.float32) + bm_ref[...]
    ls_ref[...] = jnp.dot(h2, wl_ref[...], preferred_element_type=jnp.float32) + bl_ref[...]


def _k4(sagg, ha, hb, dinv, wm, bm, wl, bl):
    g = NPAD // _RB
    half_spec = pl.BlockSpec((_RB, HALF), lambda i: (i, 0))
    sa_spec = pl.BlockSpec((1, _RB, HALF), lambda i: (0, i, 0))
    sb_spec = pl.BlockSpec((1, _RB, HALF), lambda i: (1, i, 0))
    lat_spec = pl.BlockSpec((_RB, N_LAT), lambda i: (i, 0))
    return pl.pallas_call(
        _k4_body,
        grid=(g,),
        in_specs=[
            sa_spec, sb_spec, half_spec, half_spec,
            pl.BlockSpec((_RB, 1), lambda i: (i, 0)),
            pl.BlockSpec((N_HID, N_LAT), lambda i: (0, 0)),
            pl.BlockSpec((1, N_LAT), lambda i: (0, 0)),
            pl.BlockSpec((N_HID, N_LAT), lambda i: (0, 0)),
            pl.BlockSpec((1, N_LAT), lambda i: (0, 0)),
        ],
        out_specs=[lat_spec, lat_spec],
        out_shape=[
            jax.ShapeDtypeStruct((N_NODES, N_LAT), jnp.float32),
            jax.ShapeDtypeStruct((N_NODES, N_LAT), jnp.float32),
        ],
    )(sagg, sagg, ha, hb, dinv, wm, bm, wl, bl)


def kernel(x, edge_index, W_fc, b_fc, W1, b1, W_mu, b_mu, W_ls, b_ls):
    _deg_sc, _spmm_sc = _sc_kernels()

    if E_SPMM == N_EDGES:
        src = edge_index[0]
        dst = edge_index[1]
    else:
        pad_s = jnp.full((E_SPMM - N_EDGES,), PAD_NODE, dtype=edge_index.dtype)
        src = jnp.concatenate([edge_index[0], pad_s])
        dst = jnp.concatenate([edge_index[1], pad_s])
    pad_d = jnp.full((E_PAD - N_EDGES,), PAD_NODE, dtype=edge_index.dtype)
    dst2d = jnp.concatenate([edge_index[1], pad_d]).reshape(E_PAD // DCH, DCH)

    d = _deg_sc(dst2d)
    d0 = d[0].reshape(NPAD, 1)
    d1 = d[1].reshape(NPAD, 1)

    zseg = jnp.zeros((SEG, HALF), dtype=jnp.float32)
    dinv, hpa, hpb = _k1(x, W_fc, b_fc.reshape(1, -1), d0, d1)
    s1 = _spmm_sc(src, dst, hpa, hpb, zseg)
    h2pa, h2pb = _k3(s1, hpa, hpb, dinv, W1, b1.reshape(1, -1))
    s2 = _spmm_sc(src, dst, h2pa, h2pb, zseg)
    mu, ls = _k4(s2, h2pa, h2pb, dinv,
                 W_mu, b_mu.reshape(1, -1), W_ls, b_ls.reshape(1, -1))
    return (mu, ls)
